# Initial kernel scaffold; baseline (speedup 1.0000x reference)
import jax
import jax.numpy as jnp
from jax.experimental import pallas as pl


def kernel(h, W1, b1, Wa, ba, Wb, bb, Wc, bc, Wcls0, bcls0, Wcls1, bcls1):
    raise NotImplementedError("write your pallas kernel here")


# speedup vs baseline: 1.2569x; 1.2569x over previous
import functools

import jax
import jax.numpy as jnp
from jax.experimental import pallas as pl
from jax.experimental.pallas import tpu as pltpu


def _body(N, h_ref, W1_ref, b1_ref, Wa_ref, ba_ref, Wb_ref, bb_ref, Wc_ref,
          bc_ref, Wcls_ref, bcls_ref,
          A_ref, logits_ref, yprob_ref, yhat_ref,
          m_ref, s_ref, macc_ref):
    i = pl.program_id(0)
    nb = pl.num_programs(0)
    BLK = h_ref.shape[0]
    row = i * BLK + jax.lax.broadcasted_iota(jnp.int32, (BLK, 1), 0)
    valid = row < N

    @pl.when(i == 0)
    def _init():
        m_ref[...] = jnp.full(m_ref.shape, -jnp.inf, jnp.float32)
        s_ref[...] = jnp.zeros(s_ref.shape, jnp.float32)
        macc_ref[...] = jnp.zeros(macc_ref.shape, jnp.float32)

    x = jax.lax.dot_general(h_ref[...], W1_ref[...], (((1,), (1,)), ((), ())),
                            preferred_element_type=jnp.float32)
    x = jnp.maximum(x + b1_ref[...], 0.0)
    x = jnp.where(valid, x, 0.0)
    a = jnp.tanh(jax.lax.dot_general(x, Wa_ref[...], (((1,), (1,)), ((), ())),
                                     preferred_element_type=jnp.float32)
                 + ba_ref[...])
    g = jax.nn.sigmoid(jax.lax.dot_general(x, Wb_ref[...], (((1,), (1,)), ((), ())),
                                           preferred_element_type=jnp.float32)
                       + bb_ref[...])
    A_blk = jax.lax.dot_general(a * g, Wc_ref[...], (((1,), (1,)), ((), ())),
                                preferred_element_type=jnp.float32)
    A_blk = A_blk + bc_ref[...]
    A_ref[...] = A_blk.T
    A_blk = jnp.where(valid, A_blk, -jnp.inf)

    m_old = m_ref[...]
    m_new = jnp.maximum(m_old, jnp.max(A_blk, axis=0)[None, :])
    c = jnp.exp(m_old - m_new)
    p = jnp.exp(A_blk - m_new)
    s_ref[...] = s_ref[...] * c + jnp.sum(p, axis=0)[None, :]
    contrib = jax.lax.dot_general(p, x, (((0,), (0,)), ((), ())),
                                  preferred_element_type=jnp.float32)
    macc_ref[...] = macc_ref[...] * c.T + contrib
    m_ref[...] = m_new

    @pl.when(i == nb - 1)
    def _fin():
        M = macc_ref[...] / s_ref[...].T
        logits = jnp.sum(M * Wcls_ref[...], axis=1)[None, :] + bcls_ref[...]
        logits_ref[...] = logits
        mx = jnp.max(logits, axis=1, keepdims=True)
        e = jnp.exp(logits - mx)
        yprob_ref[...] = e / jnp.sum(e, axis=1, keepdims=True)
        l0 = logits[0, 0]
        l1 = logits[0, 1]
        yhat_ref[...] = jnp.where(l1 > l0, jnp.int32(1),
                                  jnp.int32(0)).reshape(1, 1)


def kernel(h, W1, b1, Wa, ba, Wb, bb, Wc, bc, Wcls0, bcls0, Wcls1, bcls1):
    N, D = h.shape
    L = W1.shape[0]
    Dm = Wa.shape[0]
    BLK = 2048
    nb = -(-N // BLK)

    Wcls = jnp.concatenate([Wcls0, Wcls1], axis=0)
    bcls = jnp.stack([bcls0[0], bcls1[0]])[None, :]

    full = lambda shape: pl.BlockSpec(shape, lambda i: (0,) * len(shape))
    out_shapes = (
        jax.ShapeDtypeStruct((2, N), jnp.float32),
        jax.ShapeDtypeStruct((1, 2), jnp.float32),
        jax.ShapeDtypeStruct((1, 2), jnp.float32),
        jax.ShapeDtypeStruct((1, 1), jnp.int32),
    )
    A_raw, logits, y_prob, y_hat = pl.pallas_call(
        functools.partial(_body, N),
        grid=(nb,),
        in_specs=[
            pl.BlockSpec((BLK, D), lambda i: (i, 0)),
            full((L, D)),
            full((1, L)),
            full((Dm, L)),
            full((1, Dm)),
            full((Dm, L)),
            full((1, Dm)),
            full((2, Dm)),
            full((1, 2)),
            full((2, L)),
            full((1, 2)),
        ],
        out_specs=(
            pl.BlockSpec((2, BLK), lambda i: (0, i)),
            full((1, 2)),
            full((1, 2)),
            full((1, 1)),
        ),
        out_shape=out_shapes,
        scratch_shapes=[
            pltpu.VMEM((1, 2), jnp.float32),
            pltpu.VMEM((1, 2), jnp.float32),
            pltpu.VMEM((2, L), jnp.float32),
        ],
        compiler_params=pltpu.CompilerParams(
            dimension_semantics=("arbitrary",),
        ),
    )(h, W1, b1[None, :], Wa, ba[None, :], Wb, bb[None, :], Wc, bc[None, :],
      Wcls, bcls)
    return (logits, y_prob, y_hat, A_raw)


# speedup vs baseline: 1.2947x; 1.0301x over previous
import functools

import jax
import jax.numpy as jnp
from jax.experimental import pallas as pl
from jax.experimental.pallas import tpu as pltpu


def _body(N, h_ref, W1_ref, b1_ref, Wa_ref, ba_ref, Wb_ref, bb_ref, Wc_ref,
          bc_ref, Wcls_ref, bcls_ref,
          A_ref, logits_ref, yprob_ref, yhat_ref,
          m_ref, s_ref, macc_ref):
    i = pl.program_id(0)
    nb = pl.num_programs(0)
    BLK = h_ref.shape[0]
    row = i * BLK + jax.lax.broadcasted_iota(jnp.int32, (BLK, 1), 0)
    valid = row < N

    @pl.when(i == 0)
    def _init():
        m_ref[...] = jnp.full(m_ref.shape, -jnp.inf, jnp.float32)
        s_ref[...] = jnp.zeros(s_ref.shape, jnp.float32)
        macc_ref[...] = jnp.zeros(macc_ref.shape, jnp.float32)

    bf = jnp.bfloat16
    x = jax.lax.dot_general(h_ref[...].astype(bf), W1_ref[...].astype(bf),
                            (((1,), (1,)), ((), ())),
                            preferred_element_type=jnp.float32)
    x = jnp.maximum(x + b1_ref[...], 0.0)
    x = jnp.where(valid, x, 0.0)
    xb = x.astype(bf)
    a = jnp.tanh(jax.lax.dot_general(xb, Wa_ref[...].astype(bf),
                                     (((1,), (1,)), ((), ())),
                                     preferred_element_type=jnp.float32)
                 + ba_ref[...])
    g = jax.nn.sigmoid(jax.lax.dot_general(xb, Wb_ref[...].astype(bf),
                                           (((1,), (1,)), ((), ())),
                                           preferred_element_type=jnp.float32)
                       + bb_ref[...])
    A_blk = jax.lax.dot_general(a * g, Wc_ref[...], (((1,), (1,)), ((), ())),
                                preferred_element_type=jnp.float32)
    A_blk = A_blk + bc_ref[...]
    A_ref[...] = A_blk.T
    A_blk = jnp.where(valid, A_blk, -jnp.inf)

    m_old = m_ref[...]
    m_new = jnp.maximum(m_old, jnp.max(A_blk, axis=0)[None, :])
    c = jnp.exp(m_old - m_new)
    p = jnp.exp(A_blk - m_new)
    s_ref[...] = s_ref[...] * c + jnp.sum(p, axis=0)[None, :]
    contrib = jax.lax.dot_general(p, x, (((0,), (0,)), ((), ())),
                                  preferred_element_type=jnp.float32)
    macc_ref[...] = macc_ref[...] * c.T + contrib
    m_ref[...] = m_new

    @pl.when(i == nb - 1)
    def _fin():
        M = macc_ref[...] / s_ref[...].T
        logits = jnp.sum(M * Wcls_ref[...], axis=1)[None, :] + bcls_ref[...]
        logits_ref[...] = logits
        mx = jnp.max(logits, axis=1, keepdims=True)
        e = jnp.exp(logits - mx)
        yprob_ref[...] = e / jnp.sum(e, axis=1, keepdims=True)
        l0 = logits[0, 0]
        l1 = logits[0, 1]
        yhat_ref[...] = jnp.where(l1 > l0, jnp.int32(1),
                                  jnp.int32(0)).reshape(1, 1)


def kernel(h, W1, b1, Wa, ba, Wb, bb, Wc, bc, Wcls0, bcls0, Wcls1, bcls1):
    N, D = h.shape
    L = W1.shape[0]
    Dm = Wa.shape[0]
    BLK = 2048
    nb = -(-N // BLK)

    Wcls = jnp.concatenate([Wcls0, Wcls1], axis=0)
    bcls = jnp.stack([bcls0[0], bcls1[0]])[None, :]

    full = lambda shape: pl.BlockSpec(shape, lambda i: (0,) * len(shape))
    out_shapes = (
        jax.ShapeDtypeStruct((2, N), jnp.float32),
        jax.ShapeDtypeStruct((1, 2), jnp.float32),
        jax.ShapeDtypeStruct((1, 2), jnp.float32),
        jax.ShapeDtypeStruct((1, 1), jnp.int32),
    )
    A_raw, logits, y_prob, y_hat = pl.pallas_call(
        functools.partial(_body, N),
        grid=(nb,),
        in_specs=[
            pl.BlockSpec((BLK, D), lambda i: (i, 0)),
            full((L, D)),
            full((1, L)),
            full((Dm, L)),
            full((1, Dm)),
            full((Dm, L)),
            full((1, Dm)),
            full((2, Dm)),
            full((1, 2)),
            full((2, L)),
            full((1, 2)),
        ],
        out_specs=(
            pl.BlockSpec((2, BLK), lambda i: (0, i)),
            full((1, 2)),
            full((1, 2)),
            full((1, 1)),
        ),
        out_shape=out_shapes,
        scratch_shapes=[
            pltpu.VMEM((1, 2), jnp.float32),
            pltpu.VMEM((1, 2), jnp.float32),
            pltpu.VMEM((2, L), jnp.float32),
        ],
        compiler_params=pltpu.CompilerParams(
            dimension_semantics=("arbitrary",),
        ),
    )(h, W1, b1[None, :], Wa, ba[None, :], Wb, bb[None, :], Wc, bc[None, :],
      Wcls, bcls)
    return (logits, y_prob, y_hat, A_raw)
